# baseline (speedup 1.0000x reference)
import functools

import jax
import jax.numpy as jnp
from jax import lax
from jax.experimental import pallas as pl
from jax.experimental.pallas import tpu as pltpu
from jax.experimental.pallas import tpu_sc as plsc

_L = 512
_R = 1024
_V = 1024
_NK = 3125
_NW = 32
_SPW = _L // _NW


def _body(ep_hbm, res_map_hbm, res_counts_hbm, emb_hbm, sw_hbm, out_hbm,
          ep_v, cnt_v, rows_v, emb_v, sw_v, out_v,
          sem_ep, sem_cnt, sem_rows, sem_io):
    cid = lax.axis_index("c")
    sid = lax.axis_index("s")
    wid = sid * 2 + cid
    base = wid * _SPW

    ep_cp = pltpu.async_copy(ep_hbm.at[pl.ds(base, _SPW)], ep_v, sem_ep)
    emb_cp = pltpu.async_copy(emb_hbm, emb_v, sem_io)
    sw_cp = pltpu.async_copy(sw_hbm.at[pl.ds(base, _SPW)], sw_v, sem_io)
    ep_cp.wait()
    cnt_cp = pltpu.async_copy(res_counts_hbm.at[ep_v], cnt_v, sem_cnt)
    rows_cp = pltpu.async_copy(res_map_hbm.at[ep_v], rows_v, sem_rows)
    emb_cp.wait()
    sw_cp.wait()
    cnt_cp.wait()
    rows_cp.wait()

    lanes = lax.iota(jnp.int32, 16)
    cnt = cnt_v[...]

    @plsc.parallel_loop(0, _SPW, unroll=2, carry=jnp.zeros((16,), jnp.float32))
    def sums(si, sums):
        cnt_b = plsc.load_gather(cnt_v, [jnp.full((16,), si, jnp.int32)])
        cnt_s = jnp.max(cnt_b)
        nch = (cnt_s + 15) >> 4

        def chunk(j, acc):
            idx = rows_v[si, pl.ds(j * 16, 16)]
            vals = plsc.load_gather(emb_v, [idx])
            m = (j * 16 + lanes) < cnt_s
            return acc + jnp.where(m, vals, jnp.float32(0.0))

        acc = lax.fori_loop(0, nch, chunk, jnp.zeros((16,), jnp.float32))
        return jnp.where(lanes == si, jnp.sum(acc), sums)

    avg_v = sums / cnt.astype(jnp.float32)
    out_v[...] = jnp.exp(avg_v + sw_v[...])
    pltpu.sync_copy(out_v, out_hbm.at[pl.ds(base, _SPW)])


@jax.jit
def _run(encoded_parent, res_map, res_counts, emb, sw):
    mesh = plsc.VectorSubcoreMesh(core_axis_name="c", subcore_axis_name="s")
    f = functools.partial(
        pl.kernel,
        out_type=jax.ShapeDtypeStruct((_L,), jnp.float32),
        mesh=mesh,
        compiler_params=pltpu.CompilerParams(needs_layout_passes=False),
        scratch_types=[
            pltpu.VMEM((_SPW,), jnp.int32),
            pltpu.VMEM((_SPW,), jnp.int32),
            pltpu.VMEM((_SPW, _R), jnp.int32),
            pltpu.VMEM((_V,), jnp.float32),
            pltpu.VMEM((_SPW,), jnp.float32),
            pltpu.VMEM((_SPW,), jnp.float32),
            pltpu.SemaphoreType.DMA,
            pltpu.SemaphoreType.DMA,
            pltpu.SemaphoreType.DMA,
            pltpu.SemaphoreType.DMA,
        ],
    )(_body)
    return f(encoded_parent, res_map, res_counts, emb, sw)


def kernel(encoded_parent, kmer_emb, site_w, res_map, res_counts):
    emb = kmer_emb.reshape(-1)
    sw = site_w.reshape(-1)
    return _run(encoded_parent, res_map, res_counts, emb, sw)


# speedup vs baseline: 1.0025x; 1.0025x over previous
import functools

import jax
import jax.numpy as jnp
from jax import lax
from jax.experimental import pallas as pl
from jax.experimental.pallas import tpu as pltpu
from jax.experimental.pallas import tpu_sc as plsc

_L = 512
_R = 1024
_V = 1024
_NK = 3125
_NW = 32
_SPW = _L // _NW


def _body(ep_hbm, res_map_hbm, res_counts_hbm, emb_hbm, sw_hbm, out_hbm,
          ep_v, cnt_v, rows_v, emb_v, sw_v, out_v,
          sem_ep, sem_cnt, sem_rows, sem_emb, sem_sw):
    cid = lax.axis_index("c")
    sid = lax.axis_index("s")
    wid = sid * 2 + cid
    base = wid * _SPW

    ep_cp = pltpu.async_copy(ep_hbm.at[pl.ds(base, _SPW)], ep_v, sem_ep)
    emb_cp = pltpu.async_copy(emb_hbm, emb_v, sem_emb)
    sw_cp = pltpu.async_copy(sw_hbm.at[pl.ds(base, _SPW)], sw_v, sem_sw)
    ep_cp.wait()
    cnt_cp = pltpu.async_copy(res_counts_hbm.at[ep_v], cnt_v, sem_cnt)
    rows_cp = pltpu.async_copy(res_map_hbm.at[ep_v], rows_v, sem_rows)
    emb_cp.wait()
    sw_cp.wait()
    cnt_cp.wait()
    rows_cp.wait()

    lanes = lax.iota(jnp.int32, 16)
    cnt = cnt_v[...]

    def site(si, sums):
        cnt_b = plsc.load_gather(cnt_v, [jnp.full((16,), si, jnp.int32)])
        cnt_s = jnp.max(cnt_b)
        nch = (cnt_s + 15) >> 4

        def chunk(j, acc):
            idx = rows_v[si, pl.ds(j * 16, 16)]
            vals = plsc.load_gather(emb_v, [idx])
            m = (j * 16 + lanes) < cnt_s
            return acc + jnp.where(m, vals, jnp.float32(0.0))

        acc = lax.fori_loop(0, nch, chunk, jnp.zeros((16,), jnp.float32))
        return jnp.where(lanes == si, jnp.sum(acc), sums)

    sums = lax.fori_loop(0, _SPW, site, jnp.zeros((16,), jnp.float32))

    avg_v = sums / cnt.astype(jnp.float32)
    out_v[...] = jnp.exp(avg_v + sw_v[...])
    pltpu.sync_copy(out_v, out_hbm.at[pl.ds(base, _SPW)])


@jax.jit
def _run(encoded_parent, res_map, res_counts, emb, sw):
    mesh = plsc.VectorSubcoreMesh(core_axis_name="c", subcore_axis_name="s")
    f = functools.partial(
        pl.kernel,
        out_type=jax.ShapeDtypeStruct((_L,), jnp.float32),
        mesh=mesh,
        compiler_params=pltpu.CompilerParams(needs_layout_passes=False),
        scratch_types=[
            pltpu.VMEM((_SPW,), jnp.int32),
            pltpu.VMEM((_SPW,), jnp.int32),
            pltpu.VMEM((_SPW, _R), jnp.int32),
            pltpu.VMEM((_V,), jnp.float32),
            pltpu.VMEM((_SPW,), jnp.float32),
            pltpu.VMEM((_SPW,), jnp.float32),
            pltpu.SemaphoreType.DMA,
            pltpu.SemaphoreType.DMA,
            pltpu.SemaphoreType.DMA,
            pltpu.SemaphoreType.DMA,
            pltpu.SemaphoreType.DMA,
        ],
    )(_body)
    return f(encoded_parent, res_map, res_counts, emb, sw)


def kernel(encoded_parent, kmer_emb, site_w, res_map, res_counts):
    emb = kmer_emb.reshape(-1)
    sw = site_w.reshape(-1)
    return _run(encoded_parent, res_map, res_counts, emb, sw)
